# Initial kernel scaffold; baseline (speedup 1.0000x reference)
import jax
import jax.numpy as jnp
from jax.experimental import pallas as pl


def kernel(real_stats, gen_stats):
    raise NotImplementedError("write your pallas kernel here")


# speedup vs baseline: 11.6536x; 11.6536x over previous
import functools

import jax
import jax.numpy as jnp
from jax import lax
from jax.experimental import pallas as pl
from jax.experimental.pallas import tpu as pltpu

_N = 4096
_M = 4096
_K = 2048
_BM = 512
_BN = 512
_JRR = _N // _BN
_JTOT = (_N + _M) // _BN
_NNK = 5

_DOT_DN = (((1,), (1,)), ((), ()))


def _body(a_ref, b_ref, out_ref, sq_buf, rsq_buf, hit_buf):
    i = pl.program_id(0)
    j = pl.program_id(1)

    a = a_ref[...]
    b = b_ref[...]
    a2 = jnp.sum(a * a, axis=1, keepdims=True)
    b2 = jnp.sum(b * b, axis=1)[None, :]
    g = lax.dot_general(a, b, _DOT_DN,
                        preferred_element_type=jnp.float32)
    sq = (a2 - 2.0 * g) + b2

    @pl.when(j < _JRR)
    def _rr_phase():
        sq_buf[:, pl.ds(j * _BN, _BN)] = sq

        @pl.when(j == _JRR - 1)
        def _extract_radii():
            cur = sq_buf[...]
            for _ in range(_NNK):
                m = jnp.min(cur, axis=1, keepdims=True)
                cur = jnp.where(cur <= m, jnp.inf, cur)
            r6 = jnp.min(cur, axis=1, keepdims=True)
            rsq_buf[...] = jnp.maximum(r6, 0.0)

    @pl.when(j >= _JRR)
    def _rg_phase():
        r = rsq_buf[...]
        colany = jnp.max((sq <= r).astype(jnp.float32), axis=0,
                         keepdims=True)
        jj = j - _JRR

        @pl.when(i == 0)
        def _init():
            hit_buf[:, pl.ds(jj * _BN, _BN)] = colany

        @pl.when(i > 0)
        def _accum():
            prev = hit_buf[:, pl.ds(jj * _BN, _BN)]
            hit_buf[:, pl.ds(jj * _BN, _BN)] = jnp.maximum(prev, colany)

        @pl.when((i == _N // _BM - 1) & (j == _JTOT - 1))
        def _finish():
            out_ref[0, 0] = jnp.sum(hit_buf[...]) * (1.0 / _M)


@functools.partial(jax.jit)
def kernel(real_stats, gen_stats):
    b_cat = jnp.concatenate([real_stats, gen_stats], axis=0)
    grid = (_N // _BM, _JTOT)
    out = pl.pallas_call(
        _body,
        grid=grid,
        in_specs=[
            pl.BlockSpec((_BM, _K), lambda i, j: (i, 0)),
            pl.BlockSpec((_BN, _K), lambda i, j: (j, 0)),
        ],
        out_specs=pl.BlockSpec(memory_space=pltpu.SMEM),
        out_shape=jax.ShapeDtypeStruct((1, 1), jnp.float32),
        scratch_shapes=[
            pltpu.VMEM((_BM, _N), jnp.float32),
            pltpu.VMEM((_BM, 1), jnp.float32),
            pltpu.VMEM((1, _M), jnp.float32),
        ],
        compiler_params=pltpu.CompilerParams(
            dimension_semantics=("arbitrary", "arbitrary"),
        ),
        interpret=False,
    )(real_stats, b_cat)
    return out[0, 0]
